# baseline (speedup 1.0000x reference)
import functools

import jax
import jax.numpy as jnp
from jax import lax
from jax.experimental import pallas as pl
from jax.experimental.pallas import tpu as pltpu
from jax.experimental.pallas import tpu_sc as plsc

_K = 512
_B = 64
_N = 8192
_L = 16
_L2 = 32
_NW = 32
_ROWS_PER_W = _B // _NW
_CHUNKS = _N // _L
_CHUNKS2 = _N // _L2
_UNROLL = 8

_GATHER_DNUMS = lax.GatherDimensionNumbers(
    offset_dims=(), collapsed_slice_dims=(0,), start_index_map=(0,))


def _rot(v, idx):
    return lax.gather(v, idx[:, None], dimension_numbers=_GATHER_DNUMS,
                      slice_sizes=(1,),
                      mode=lax.GatherScatterMode.PROMISE_IN_BOUNDS)


def _lane_sum(v):
    iota = lax.iota(jnp.int32, _L)
    for shift in (8, 4, 2, 1):
        v = v + _rot(v, (iota + shift) & (_L - 1))
    return v


def _ge_total(cnt32):
    s = _lane_sum((cnt32 & 0xFFFF) + (cnt32 >> 16))
    return _N - s


def _splat16(v32):
    return plsc.bitcast(v32 | (v32 << 16), jnp.int16)


def _body(x_hbm, out_hbm, row_a, row_b, hi_v, lo_v, lsem_a, lsem_b, ssem):
    wid = lax.axis_index("s") * 2 + lax.axis_index("c")
    ones = jnp.ones((_L,), jnp.int32)
    zeros = jnp.zeros((_L,), jnp.int32)
    kvec = jnp.full((_L,), _K, jnp.int32)

    row0 = wid * _ROWS_PER_W
    stores = []
    cp_a = pltpu.make_async_copy(x_hbm.at[row0], row_a, lsem_a)
    cp_b = pltpu.make_async_copy(x_hbm.at[row0 + 1], row_b, lsem_b)
    cp_a.start()
    cp_b.start()

    for r in range(_ROWS_PER_W):
        row_v = row_a if r == 0 else row_b
        (cp_a if r == 0 else cp_b).wait()

        with jax.named_scope("ph_prep"):
            @plsc.parallel_loop(0, _CHUNKS // 2, unroll=_UNROLL,
                                carry=jnp.int32(0))
            def _prep(i, c):
                off = i * _L2
                b0 = lax.bitcast_convert_type(jnp.abs(row_v[pl.ds(off, _L)]),
                                              jnp.int32)
                b1 = lax.bitcast_convert_type(
                    jnp.abs(row_v[pl.ds(off + _L, _L)]), jnp.int32)
                o2 = i * _L
                hi_v[pl.ds(o2, _L)] = (b0 >> 16) | (b1 & 0x7FFF0000)
                lo_v[pl.ds(o2, _L)] = ((b0 >> 1) & 0x7FFF) | ((b1 << 15)
                                                              & 0x7FFF0000)
                return c

        def make_search(arr_v):
            def step(bi, t):
                cand = t | (ones << jnp.broadcast_to(jnp.int32(14) - bi,
                                                     (_L,)))
                cand16 = _splat16(cand)

                @plsc.parallel_loop(0, _CHUNKS2, unroll=16,
                                    carry=zeros)
                def cnt(j, cnt):
                    v16 = plsc.bitcast(arr_v[pl.ds(j * _L, _L)], jnp.int16)
                    d = plsc.bitcast(v16 - cand16, jnp.int32)
                    return cnt + ((d >> 15) & 0x00010001)
                tot = _ge_total(cnt)
                return jnp.where(tot >= kvec, cand, t)
            return step

        with jax.named_scope("ph_hisearch"):
            t1 = lax.fori_loop(0, 15, make_search(hi_v), zeros)

        t16 = _splat16(t1)

        with jax.named_scope("ph_fold"):
            @plsc.parallel_loop(0, _CHUNKS2, unroll=_UNROLL,
                                carry=jnp.int32(0))
            def _fold(j, c):
                off = j * _L
                h = plsc.bitcast(hi_v[pl.ds(off, _L)], jnp.int16)
                l32 = lo_v[pl.ds(off, _L)]
                pl_ = (plsc.bitcast(h - t16, jnp.int32) >> 15) & 0x00010001
                pg = (plsc.bitcast(t16 - h, jnp.int32) >> 15) & 0x00010001
                ltm = (pl_ << 16) - pl_
                gtm = (pg << 16) - pg
                eqm = ~(ltm | gtm)
                lo_v[pl.ds(off, _L)] = (l32 & eqm) | (gtm & 0x7FFF7FFF) | ltm
                return c

        with jax.named_scope("ph_losearch"):
            t2 = lax.fori_loop(0, 15, make_search(lo_v), zeros)

        cand0 = (t1 << 16) | (t2 << 1) | 1

        with jax.named_scope("ph_lastbit"):
            @plsc.parallel_loop(0, _CHUNKS, unroll=_UNROLL, carry=zeros)
            def cnt(i, cnt):
                b = lax.bitcast_convert_type(
                    jnp.abs(row_v[pl.ds(i * _L, _L)]), jnp.int32)
                return cnt + jnp.where(b >= cand0, ones, zeros)
        tot = _lane_sum(cnt)
        thresh = jnp.where(tot >= kvec, cand0, cand0 & ~1)

        with jax.named_scope("ph_mask"):
            @plsc.parallel_loop(0, _CHUNKS, unroll=_UNROLL,
                                carry=jnp.int32(0))
            def _mask(i, c):
                v = row_v[pl.ds(i * _L, _L)]
                keep = lax.bitcast_convert_type(jnp.abs(v), jnp.int32) >= thresh
                row_v[pl.ds(i * _L, _L)] = jnp.where(keep, v, jnp.float32(0))
                return c
        st = pltpu.make_async_copy(row_v, out_hbm.at[row0 + r], ssem)
        st.start()
        stores.append(st)

    for st in stores:
        st.wait()


@jax.jit
def kernel(x):
    mesh = plsc.VectorSubcoreMesh(core_axis_name="c", subcore_axis_name="s")
    fn = functools.partial(
        pl.kernel,
        mesh=mesh,
        compiler_params=pltpu.CompilerParams(needs_layout_passes=False),
        out_type=jax.ShapeDtypeStruct((_B, _N), jnp.float32),
        scratch_types=[
            pltpu.VMEM((_N,), jnp.float32),
            pltpu.VMEM((_N,), jnp.float32),
            pltpu.VMEM((_N // 2,), jnp.int32),
            pltpu.VMEM((_N // 2,), jnp.int32),
            pltpu.SemaphoreType.DMA,
            pltpu.SemaphoreType.DMA,
            pltpu.SemaphoreType.DMA,
        ],
    )(_body)
    return fn(x)


# speedup vs baseline: 1.0141x; 1.0141x over previous
import functools

import jax
import jax.numpy as jnp
from jax import lax
from jax.experimental import pallas as pl
from jax.experimental.pallas import tpu as pltpu
from jax.experimental.pallas import tpu_sc as plsc

_K = 512
_B = 64
_N = 8192
_L = 16
_L2 = 32
_NW = 32
_ROWS_PER_W = _B // _NW
_CHUNKS = _N // _L
_CHUNKS2 = _N // _L2
_UNROLL = 8

_GATHER_DNUMS = lax.GatherDimensionNumbers(
    offset_dims=(), collapsed_slice_dims=(0,), start_index_map=(0,))


def _rot(v, idx):
    return lax.gather(v, idx[:, None], dimension_numbers=_GATHER_DNUMS,
                      slice_sizes=(1,),
                      mode=lax.GatherScatterMode.PROMISE_IN_BOUNDS)


def _lane_sum(v):
    iota = lax.iota(jnp.int32, _L)
    for shift in (8, 4, 2, 1):
        v = v + _rot(v, (iota + shift) & (_L - 1))
    return v


def _ge_total(cnt32):
    s = _lane_sum((cnt32 & 0xFFFF) + (cnt32 >> 16))
    return _N - s


def _splat16(v32):
    return plsc.bitcast(v32 | (v32 << 16), jnp.int16)


def _body(x_hbm, out_hbm, row_a, row_b, hi_v, lo_v, lsem_a, lsem_b, ssem):
    wid = lax.axis_index("s") * 2 + lax.axis_index("c")
    ones = jnp.ones((_L,), jnp.int32)
    zeros = jnp.zeros((_L,), jnp.int32)
    kvec = jnp.full((_L,), _K, jnp.int32)

    row0 = wid * _ROWS_PER_W
    stores = []
    cp_a = pltpu.make_async_copy(x_hbm.at[row0], row_a, lsem_a)
    cp_b = pltpu.make_async_copy(x_hbm.at[row0 + 1], row_b, lsem_b)
    cp_a.start()
    cp_b.start()

    for r in range(_ROWS_PER_W):
        row_v = row_a if r == 0 else row_b
        (cp_a if r == 0 else cp_b).wait()

        with jax.named_scope("ph_prep"):
            @plsc.parallel_loop(0, _CHUNKS // 2, unroll=_UNROLL,
                                carry=jnp.int32(0))
            def _prep(i, c):
                off = i * _L2
                b0 = lax.bitcast_convert_type(jnp.abs(row_v[pl.ds(off, _L)]),
                                              jnp.int32)
                b1 = lax.bitcast_convert_type(
                    jnp.abs(row_v[pl.ds(off + _L, _L)]), jnp.int32)
                o2 = i * _L
                hi_v[pl.ds(o2, _L)] = (b0 >> 16) | (b1 & 0x7FFF0000)
                lo_v[pl.ds(o2, _L)] = ((b0 >> 1) & 0x7FFF) | ((b1 << 15)
                                                              & 0x7FFF0000)
                return c

        def make_search(arr_v):
            def step(bi, t):
                cand = t | (ones << jnp.broadcast_to(jnp.int32(14) - bi,
                                                     (_L,)))
                cand16 = _splat16(cand)

                @plsc.parallel_loop(0, _CHUNKS2, unroll=_UNROLL,
                                    carry=zeros)
                def cnt(j, cnt):
                    v16 = plsc.bitcast(arr_v[pl.ds(j * _L, _L)], jnp.int16)
                    d = plsc.bitcast(v16 - cand16, jnp.int32)
                    return cnt + ((d >> 15) & 0x00010001)
                tot = _ge_total(cnt)
                return jnp.where(tot >= kvec, cand, t)
            return step

        with jax.named_scope("ph_hisearch"):
            t1 = lax.fori_loop(0, 15, make_search(hi_v), zeros)

        t16 = _splat16(t1)

        with jax.named_scope("ph_fold"):
            @plsc.parallel_loop(0, _CHUNKS2, unroll=_UNROLL,
                                carry=jnp.int32(0))
            def _fold(j, c):
                off = j * _L
                h = plsc.bitcast(hi_v[pl.ds(off, _L)], jnp.int16)
                l32 = lo_v[pl.ds(off, _L)]
                pl_ = (plsc.bitcast(h - t16, jnp.int32) >> 15) & 0x00010001
                pg = (plsc.bitcast(t16 - h, jnp.int32) >> 15) & 0x00010001
                ltm = (pl_ << 16) - pl_
                gtm = (pg << 16) - pg
                eqm = ~(ltm | gtm)
                lo_v[pl.ds(off, _L)] = (l32 & eqm) | (gtm & 0x7FFF7FFF) | ltm
                return c

        with jax.named_scope("ph_losearch"):
            t2 = lax.fori_loop(0, 15, make_search(lo_v), zeros)

        cand0 = (t1 << 16) | (t2 << 1) | 1

        with jax.named_scope("ph_lastbit"):
            @plsc.parallel_loop(0, _CHUNKS, unroll=_UNROLL, carry=zeros)
            def cnt(i, cnt):
                b = lax.bitcast_convert_type(
                    jnp.abs(row_v[pl.ds(i * _L, _L)]), jnp.int32)
                return cnt + jnp.where(b >= cand0, ones, zeros)
        tot = _lane_sum(cnt)
        thresh = jnp.where(tot >= kvec, cand0, cand0 & ~1)

        with jax.named_scope("ph_mask"):
            @plsc.parallel_loop(0, _CHUNKS, unroll=_UNROLL,
                                carry=jnp.int32(0))
            def _mask(i, c):
                v = row_v[pl.ds(i * _L, _L)]
                keep = lax.bitcast_convert_type(jnp.abs(v), jnp.int32) >= thresh
                row_v[pl.ds(i * _L, _L)] = jnp.where(keep, v, jnp.float32(0))
                return c
        st = pltpu.make_async_copy(row_v, out_hbm.at[row0 + r], ssem)
        st.start()
        stores.append(st)

    for st in stores:
        st.wait()


@jax.jit
def kernel(x):
    mesh = plsc.VectorSubcoreMesh(core_axis_name="c", subcore_axis_name="s")
    fn = functools.partial(
        pl.kernel,
        mesh=mesh,
        compiler_params=pltpu.CompilerParams(needs_layout_passes=False),
        out_type=jax.ShapeDtypeStruct((_B, _N), jnp.float32),
        scratch_types=[
            pltpu.VMEM((_N,), jnp.float32),
            pltpu.VMEM((_N,), jnp.float32),
            pltpu.VMEM((_N // 2,), jnp.int32),
            pltpu.VMEM((_N // 2,), jnp.int32),
            pltpu.SemaphoreType.DMA,
            pltpu.SemaphoreType.DMA,
            pltpu.SemaphoreType.DMA,
        ],
    )(_body)
    return fn(x)
